# baseline (speedup 1.0000x reference)
import functools

import jax
import jax.numpy as jnp
import numpy as np
from jax import lax
from jax.experimental import pallas as pl
from jax.experimental.pallas import tpu as pltpu

_B, _N, _D = 16, 4096, 128
_L, _M = 128, 1024
_LOGN = 12


def _static_interp_weights():
    xg = np.linspace(0.0, 1.0, _N + 2).astype(np.float32)[1:-1]
    xng = np.linspace(0.0, 1.0, _M + 2).astype(np.float32)[1:-1]
    ind = np.searchsorted(xg, xng, side="left")
    ind = np.clip(ind - 1, 0, _N - 2)
    eps = np.float32(np.finfo(np.float32).eps)
    t = (xng - xg[ind]) / (eps + (xg[ind + 1] - xg[ind]))
    m = np.arange(_M)
    wa = np.zeros((_M, 4), np.float32)
    c0 = ind - 4 * m
    assert np.all((c0 >= 0) & (c0 <= 2))
    np.add.at(wa, (m, c0), 1.0 - t)
    np.add.at(wa, (m, c0 + 1), t)
    ind2 = (_N - 1) - ind[::-1]
    t2 = t[::-1]
    wd = np.zeros((_M, 4), np.float32)
    d0 = ind2 - 4 * m
    assert np.all((d0 >= 1) & (d0 <= 3))
    np.add.at(wd, (m, d0), 1.0 - t2)
    np.add.at(wd, (m, d0 - 1), t2)
    return wa, wd


def _bitonic_substage(S, stage, dist, asc_lane):
    n, L = S.shape
    G = n // (2 * dist)
    A = S.reshape(G, 2, dist, L)
    lo = A[:, 0]
    hi = A[:, 1]
    mn = jnp.minimum(lo, hi)
    mx = jnp.maximum(lo, hi)
    if stage == _LOGN:
        asc = jnp.broadcast_to(asc_lane.reshape(1, 1, L), (G, dist, L))
    else:
        shift = stage - dist.bit_length()
        g = lax.broadcasted_iota(jnp.int32, (G, dist, L), 0)
        asc = ((g >> shift) & 1) == 0
    nlo = jnp.where(asc, mn, mx)
    nhi = jnp.where(asc, mx, mn)
    return jnp.concatenate([nlo[:, None], nhi[:, None]], axis=1).reshape(n, L)


def _bitonic_pass2(S, stage, d, asc_lane):
    n, L = S.shape
    G = n // (4 * d)
    A = S.reshape(G, 2, 2, d, L)
    a0, a1, a2, a3 = A[:, 0, 0], A[:, 0, 1], A[:, 1, 0], A[:, 1, 1]
    if stage == _LOGN:
        asc = jnp.broadcast_to(asc_lane.reshape(1, 1, L), (G, d, L))
    else:
        shift = stage - d.bit_length() - 1
        g = lax.broadcasted_iota(jnp.int32, (G, d, L), 0)
        asc = ((g >> shift) & 1) == 0

    def cx(u, v):
        mn = jnp.minimum(u, v)
        mx = jnp.maximum(u, v)
        return jnp.where(asc, mn, mx), jnp.where(asc, mx, mn)

    x0, x2 = cx(a0, a2)
    x1, x3 = cx(a1, a3)
    y0, y1 = cx(x0, x1)
    y2, y3 = cx(x2, x3)
    return jnp.concatenate(
        [y0[:, None], y1[:, None], y2[:, None], y3[:, None]], axis=1
    ).reshape(n, L)


def _bitonic_sort(S, asc_lane):
    for stage in range(1, _LOGN + 1):
        dists = [1 << k for k in range(stage - 1, -1, -1)]
        i = 0
        while i < len(dists):
            if i + 1 < len(dists):
                S = _bitonic_pass2(S, stage, dists[i + 1], asc_lane)
                i += 2
            else:
                S = _bitonic_substage(S, stage, dists[i], asc_lane)
                i += 1
    return S


def _swe_body(x_ref, th_ref, rp_ref, wa_ref, wd_ref, out_ref):
    x = x_ref[0]
    th = th_ref[...]
    w = th * lax.rsqrt(jnp.sum(th * th, axis=1, keepdims=True))
    dn = (((1,), (1,)), ((), ()))
    S = lax.dot_general(x, w, dn, preferred_element_type=jnp.float32)
    Rs = lax.dot_general(rp_ref[...], w, dn,
                         preferred_element_type=jnp.float32)
    asc_lane = (Rs[_M - 1:_M, :] - Rs[0:1, :]) >= 0.0

    S = _bitonic_sort(S, asc_lane)

    C = S.reshape(_M, 4, _L)
    Ya = jnp.zeros((_M, _L), jnp.float32)
    Yd = jnp.zeros((_M, _L), jnp.float32)
    for c in range(4):
        Ya = Ya + wa_ref[:, c:c + 1] * C[:, c, :]
        Yd = Yd + wd_ref[:, c:c + 1] * C[:, c, :]
    rf = rp_ref[:, 0:1].astype(jnp.bfloat16).astype(jnp.float32)
    rf_up = jnp.concatenate([rf[1:], rf[_M - 1:] + 2.0], axis=0)
    rf_dn = jnp.concatenate([rf[:1] - 2.0, rf[:_M - 1]], axis=0)
    ta = rf == rf_up
    tb = rf == rf_dn
    yd_up = jnp.concatenate([Yd[1:], Yd[_M - 1:]], axis=0)
    yd_dn = jnp.concatenate([Yd[:1], Yd[:_M - 1]], axis=0)
    Yd = jnp.where(ta, yd_up, jnp.where(tb, yd_dn, Yd))
    Y = jnp.where(asc_lane, Ya, Yd)
    out_ref[0] = (Rs - Y).T


@jax.jit
def kernel(X, theta_v, ref_points):
    wa, wd = _static_interp_weights()
    wa = jnp.asarray(wa)
    wd = jnp.asarray(wd)
    out = pl.pallas_call(
        _swe_body,
        grid=(_B,),
        in_specs=[
            pl.BlockSpec((1, _N, _D), lambda b: (b, 0, 0)),
            pl.BlockSpec((_L, _D), lambda b: (0, 0)),
            pl.BlockSpec((_M, _D), lambda b: (0, 0)),
            pl.BlockSpec((_M, 4), lambda b: (0, 0)),
            pl.BlockSpec((_M, 4), lambda b: (0, 0)),
        ],
        out_specs=pl.BlockSpec((1, _L, _M), lambda b: (b, 0, 0)),
        out_shape=jax.ShapeDtypeStruct((_B, _L, _M), jnp.float32),
    )(X, theta_v, ref_points, wa, wd)
    return out.reshape(_B, _L * _M)


# speedup vs baseline: 1.2902x; 1.2902x over previous
import functools

import jax
import jax.numpy as jnp
import numpy as np
from jax import lax
from jax.experimental import pallas as pl
from jax.experimental.pallas import tpu as pltpu

_B, _N, _D = 16, 4096, 128
_L, _M = 128, 1024
_LOGN = 12


def _static_interp_weights():
    xg = np.linspace(0.0, 1.0, _N + 2).astype(np.float32)[1:-1]
    xng = np.linspace(0.0, 1.0, _M + 2).astype(np.float32)[1:-1]
    ind = np.searchsorted(xg, xng, side="left")
    ind = np.clip(ind - 1, 0, _N - 2)
    eps = np.float32(np.finfo(np.float32).eps)
    t = (xng - xg[ind]) / (eps + (xg[ind + 1] - xg[ind]))
    m = np.arange(_M)
    wa = np.zeros((_M, 4), np.float32)
    c0 = ind - 4 * m
    assert np.all((c0 >= 0) & (c0 <= 2))
    np.add.at(wa, (m, c0), 1.0 - t)
    np.add.at(wa, (m, c0 + 1), t)
    ind2 = (_N - 1) - ind[::-1]
    t2 = t[::-1]
    wd = np.zeros((_M, 4), np.float32)
    d0 = ind2 - 4 * m
    assert np.all((d0 >= 1) & (d0 <= 3))
    np.add.at(wd, (m, d0), 1.0 - t2)
    np.add.at(wd, (m, d0 - 1), t2)
    return wa, wd


def _bitonic_substage(S, stage, dist, asc_lane):
    n, L = S.shape
    G = n // (2 * dist)
    A = S.reshape(G, 2, dist, L)
    lo = A[:, 0]
    hi = A[:, 1]
    mn = jnp.minimum(lo, hi)
    mx = jnp.maximum(lo, hi)
    if stage == _LOGN:
        asc = jnp.broadcast_to(asc_lane.reshape(1, 1, L), (G, dist, L))
    else:
        shift = stage - dist.bit_length()
        g = lax.broadcasted_iota(jnp.int32, (G, dist, L), 0)
        asc = ((g >> shift) & 1) == 0
    nlo = jnp.where(asc, mn, mx)
    nhi = jnp.where(asc, mx, mn)
    return jnp.concatenate([nlo[:, None], nhi[:, None]], axis=1).reshape(n, L)


def _bitonic_pass2(S, stage, d, asc_lane):
    n, L = S.shape
    G = n // (4 * d)
    A = S.reshape(G, 2, 2, d, L)
    a0, a1, a2, a3 = A[:, 0, 0], A[:, 0, 1], A[:, 1, 0], A[:, 1, 1]
    if stage == _LOGN:
        asc = jnp.broadcast_to(asc_lane.reshape(1, 1, L), (G, d, L))
    else:
        shift = stage - d.bit_length() - 1
        g = lax.broadcasted_iota(jnp.int32, (G, d, L), 0)
        asc = ((g >> shift) & 1) == 0

    def cx(u, v):
        mn = jnp.minimum(u, v)
        mx = jnp.maximum(u, v)
        return jnp.where(asc, mn, mx), jnp.where(asc, mx, mn)

    x0, x2 = cx(a0, a2)
    x1, x3 = cx(a1, a3)
    y0, y1 = cx(x0, x1)
    y2, y3 = cx(x2, x3)
    return jnp.concatenate(
        [y0[:, None], y1[:, None], y2[:, None], y3[:, None]], axis=1
    ).reshape(n, L)


def _bitonic_sort(S, asc_lane):
    for stage in range(1, _LOGN + 1):
        dists = [1 << k for k in range(stage - 1, -1, -1)]
        i = 0
        while i < len(dists):
            if i + 1 < len(dists):
                S = _bitonic_pass2(S, stage, dists[i + 1], asc_lane)
                i += 2
            else:
                S = _bitonic_substage(S, stage, dists[i], asc_lane)
                i += 1
    return S


def _swe_body(x_ref, th_ref, rp_ref, wa_ref, wd_ref, out_ref):
    x = x_ref[0]
    th = th_ref[...]
    w = th * lax.rsqrt(jnp.sum(th * th, axis=1, keepdims=True))
    dn = (((1,), (1,)), ((), ()))
    S = lax.dot_general(x, w, dn, preferred_element_type=jnp.float32,
                        precision=lax.Precision.HIGHEST)
    Rs = lax.dot_general(rp_ref[...], w, dn,
                         preferred_element_type=jnp.float32)
    asc_lane = (Rs[_M - 1:_M, :] - Rs[0:1, :]) >= 0.0

    S = _bitonic_sort(S, asc_lane)

    C = S.reshape(_M, 4, _L)
    Ya = jnp.zeros((_M, _L), jnp.float32)
    Yd = jnp.zeros((_M, _L), jnp.float32)
    for c in range(4):
        Ya = Ya + wa_ref[:, c:c + 1] * C[:, c, :]
        Yd = Yd + wd_ref[:, c:c + 1] * C[:, c, :]
    rf = rp_ref[:, 0:1].astype(jnp.bfloat16).astype(jnp.float32)
    rf_up = jnp.concatenate([rf[1:], rf[_M - 1:] + 2.0], axis=0)
    rf_dn = jnp.concatenate([rf[:1] - 2.0, rf[:_M - 1]], axis=0)
    ta = rf == rf_up
    tb = rf == rf_dn
    yd_up = jnp.concatenate([Yd[1:], Yd[_M - 1:]], axis=0)
    yd_dn = jnp.concatenate([Yd[:1], Yd[:_M - 1]], axis=0)
    Yd = jnp.where(ta, yd_up, jnp.where(tb, yd_dn, Yd))
    Y = jnp.where(asc_lane, Ya, Yd)
    out_ref[0] = (Rs - Y).T


@jax.jit
def kernel(X, theta_v, ref_points):
    wa, wd = _static_interp_weights()
    wa = jnp.asarray(wa)
    wd = jnp.asarray(wd)
    out = pl.pallas_call(
        _swe_body,
        grid=(_B,),
        in_specs=[
            pl.BlockSpec((1, _N, _D), lambda b: (b, 0, 0)),
            pl.BlockSpec((_L, _D), lambda b: (0, 0)),
            pl.BlockSpec((_M, _D), lambda b: (0, 0)),
            pl.BlockSpec((_M, 4), lambda b: (0, 0)),
            pl.BlockSpec((_M, 4), lambda b: (0, 0)),
        ],
        out_specs=pl.BlockSpec((1, _L, _M), lambda b: (b, 0, 0)),
        out_shape=jax.ShapeDtypeStruct((_B, _L, _M), jnp.float32),
    )(X, theta_v, ref_points, wa, wd)
    return out.reshape(_B, _L * _M)
